# baseline (speedup 1.0000x reference)
import functools

import jax
import jax.numpy as jnp
from jax.experimental import pallas as pl

WCLS = 1.0
WEMB = 0.1
DIRTY_FRAC = 0.02
TOO_SIMPLE_FRAC = 0.1

_INT_MIN = -(2 ** 31)
_INT_MAX = 2 ** 31 - 1


def _ce_body(x_ref, lab_ref, o_ref):
    x = x_ref[...]
    lab = lab_ref[0, 0, :]
    m = jnp.max(x, axis=1, keepdims=True)
    logz = jnp.log(jnp.sum(jnp.exp(x - m), axis=1)) + m[:, 0]
    br, c = x.shape
    iota = jax.lax.broadcasted_iota(jnp.int32, (br, c), 1)
    picked = jnp.sum(jnp.where(iota == lab[:, None], x, 0.0), axis=1)
    o_ref[0, 0, :] = logz - picked


def _sortable_key(x):
    b = jax.lax.bitcast_convert_type(x, jnp.int32)
    return jnp.where(b >= 0, b, jnp.int32(_INT_MIN) - b)


def _key_to_float(t):
    b = jnp.where(t >= 0, t, jnp.int32(_INT_MIN) - t)
    return jax.lax.bitcast_convert_type(b, jnp.float32)


def _kth_smallest_key(s, k):
    def body(_, lohi):
        lo, hi = lohi
        mid = (lo & hi) + ((lo ^ hi) >> 1)
        c = jnp.sum((s <= mid).astype(jnp.int32))
        take = c >= k
        return (jnp.where(take, lo, mid + 1), jnp.where(take, mid, hi))

    lo, _ = jax.lax.fori_loop(0, 32, body, (jnp.int32(_INT_MIN), jnp.int32(_INT_MAX)))
    return lo


def _finish_body(cls_ref, p_ref, pt_ref, o_ref, *, n, tpk, dk):
    g = jax.lax.dot_general(
        p_ref[...], pt_ref[...], (((1,), (0,)), ((), ())),
        precision=jax.lax.Precision.HIGHEST,
        preferred_element_type=jnp.float32)
    proto_loss = jnp.sum(jnp.maximum(g - 0.14, 0.0)) / float(g.size)

    cls = cls_ref[...]
    s = _sortable_key(cls)

    t1 = _kth_smallest_key(s, tpk)
    t2 = _kth_smallest_key(s, n - dk + 1)
    t1f = _key_to_float(t1)
    t2f = _key_to_float(t2)

    cnt_lt1 = jnp.sum((s < t1).astype(jnp.int32))
    m1 = (tpk - cnt_lt1).astype(jnp.float32)
    restore1 = (t1f <= 0.5).astype(jnp.float32)
    mask_e = (s < t1) & (cls <= 0.5)
    easy_cnt = jnp.sum(mask_e.astype(jnp.float32)) + m1 * restore1
    easy_sum = jnp.sum(jnp.where(mask_e, cls, 0.0)) + m1 * t1f * restore1

    mask_d = s > t2
    cnt_gt2 = jnp.sum(mask_d.astype(jnp.int32))
    m2 = (dk - cnt_gt2).astype(jnp.float32)
    dirty_sum = jnp.sum(jnp.where(mask_d, cls, 0.0)) + m2 * t2f

    total = jnp.sum(cls)
    weighted = total - easy_sum - dirty_sum
    sum_w = float(n) - easy_cnt - float(dk)
    red = weighted / (sum_w + 1e-05)
    loss = red * WCLS + WEMB * proto_loss

    sub = jax.lax.broadcasted_iota(jnp.int32, (8, 128), 0)
    lane = jax.lax.broadcasted_iota(jnp.int32, (8, 128), 1)
    v = jnp.where((sub == 0) & (lane == 0), loss, 0.0)
    v = jnp.where((sub == 0) & (lane == 1), red, v)
    v = jnp.where((sub == 0) & (lane == 2), proto_loss, v)
    o_ref[...] = v


def kernel(proto, outcls, label_flatten):
    n, c = outcls.shape
    label = label_flatten.astype(jnp.int32)
    tpk = int(n * TOO_SIMPLE_FRAC)
    dk = int(n * DIRTY_FRAC)

    br = 512
    nb = n // br
    label3 = label.reshape(nb, 1, br)
    clsloss = pl.pallas_call(
        _ce_body,
        grid=(nb,),
        in_specs=[
            pl.BlockSpec((br, c), lambda i: (i, 0)),
            pl.BlockSpec((1, 1, br), lambda i: (i, 0, 0)),
        ],
        out_specs=pl.BlockSpec((1, 1, br), lambda i: (i, 0, 0)),
        out_shape=jax.ShapeDtypeStruct((nb, 1, br), jnp.float32),
    )(outcls, label3)

    cls2 = clsloss.reshape(128, n // 128)
    p = proto[1:]
    pt = p.T

    out = pl.pallas_call(
        functools.partial(_finish_body, n=n, tpk=tpk, dk=dk),
        in_specs=[
            pl.BlockSpec(cls2.shape, lambda: (0, 0)),
            pl.BlockSpec(p.shape, lambda: (0, 0)),
            pl.BlockSpec(pt.shape, lambda: (0, 0)),
        ],
        out_specs=pl.BlockSpec((8, 128), lambda: (0, 0)),
        out_shape=jax.ShapeDtypeStruct((8, 128), jnp.float32),
    )(cls2, p, pt)

    loss = out[0, 0]
    terms = out[0, 0:3]
    return loss, terms
